# Initial kernel scaffold; baseline (speedup 1.0000x reference)
import functools

import jax
import jax.numpy as jnp
from jax import lax
from jax.experimental import pallas as pl
from jax.experimental.pallas import tpu as pltpu
from jax.experimental.pallas import tpu_sc as plsc

N = 10000
E = 320000
F_IN = 128
H = 32
G = 64
C = 10

NP = 10240
NCORE = 2
NSUB = 16
NW = NCORE * NSUB
EPW = E // NW
B = 80
NCHUNK = EPW // B
RPS = NP // NSUB

_mesh = plsc.VectorSubcoreMesh(core_axis_name="c", subcore_axis_name="s")

_GDN = lax.GatherDimensionNumbers(
    offset_dims=(), collapsed_slice_dims=(0,), start_index_map=(0,))


def _lane_bcast(vec, i):
    idx = jnp.full((16,), i, jnp.int32)
    return lax.gather(vec, idx[:, None], _GDN, slice_sizes=(1,),
                      mode=lax.GatherScatterMode.PROMISE_IN_BOUNDS)


def _zero_shared(buf, sh, sid, width16):
    zv = jnp.zeros((16,), jnp.float32)
    for r in range(B):
        for k in range(width16):
            buf[r, pl.ds(16 * k, 16)] = zv
    @pl.loop(0, RPS // B)
    def _(k):
        pltpu.sync_copy(buf, sh.at[pl.ds(sid * RPS + k * B, B)])


@functools.partial(
    pl.kernel, mesh=_mesh,
    out_type=jax.ShapeDtypeStruct((NCORE, NP, 16), jnp.float32),
    scratch_types=[
        pltpu.VMEM((NCHUNK, B), jnp.int32),
        pltpu.VMEM((NCHUNK, B), jnp.float32),
        pltpu.VMEM((B, 16), jnp.float32),
        pltpu.VMEM_SHARED((NP, 16), jnp.float32),
    ],
)
def _deg_call(dst2d, ew2d, out, dst_v, ew_v, rows, deg_sh):
    cid = lax.axis_index("c")
    sid = lax.axis_index("s")
    wid = cid * NSUB + sid
    _zero_shared(rows, deg_sh, sid, 1)
    plsc.subcore_barrier()
    pltpu.sync_copy(dst2d.at[pl.ds(wid * NCHUNK, NCHUNK)], dst_v)
    pltpu.sync_copy(ew2d.at[pl.ds(wid * NCHUNK, NCHUNK)], ew_v)

    @pl.loop(0, NCHUNK)
    def _(ci):
        for j in range(B // 16):
            wv = ew_v[ci, pl.ds(16 * j, 16)]
            for e in range(16):
                rows[16 * j + e, pl.ds(0, 16)] = _lane_bcast(wv, e)
        pltpu.sync_copy(rows, deg_sh.at[dst_v.at[ci]], add=True)

    plsc.subcore_barrier()
    pltpu.sync_copy(deg_sh.at[pl.ds(sid * RPS, RPS)],
                    out.at[cid].at[pl.ds(sid * RPS, RPS)])


@functools.partial(
    pl.kernel, mesh=_mesh,
    out_type=jax.ShapeDtypeStruct((NCORE, NP, H), jnp.float32),
    scratch_types=[
        pltpu.VMEM((NCHUNK, B), jnp.int32),
        pltpu.VMEM((NCHUNK, B), jnp.int32),
        pltpu.VMEM((NCHUNK, B), jnp.float32),
        pltpu.VMEM((B, H), jnp.float32),
        pltpu.VMEM_SHARED((NP, H), jnp.float32),
    ],
)
def _agg_call(hs, src2d, dst2d, ew2d, out, src_v, dst_v, ew_v, rows, agg_sh):
    cid = lax.axis_index("c")
    sid = lax.axis_index("s")
    wid = cid * NSUB + sid
    _zero_shared(rows, agg_sh, sid, H // 16)
    plsc.subcore_barrier()
    pltpu.sync_copy(src2d.at[pl.ds(wid * NCHUNK, NCHUNK)], src_v)
    pltpu.sync_copy(dst2d.at[pl.ds(wid * NCHUNK, NCHUNK)], dst_v)
    pltpu.sync_copy(ew2d.at[pl.ds(wid * NCHUNK, NCHUNK)], ew_v)

    @pl.loop(0, NCHUNK)
    def _(ci):
        pltpu.sync_copy(hs.at[src_v.at[ci]], rows)
        for j in range(B // 16):
            wv = ew_v[ci, pl.ds(16 * j, 16)]
            for e in range(16):
                r = 16 * j + e
                w = _lane_bcast(wv, e)
                rows[r, pl.ds(0, 16)] = rows[r, pl.ds(0, 16)] * w
                rows[r, pl.ds(16, 16)] = rows[r, pl.ds(16, 16)] * w
        pltpu.sync_copy(rows, agg_sh.at[dst_v.at[ci]], add=True)

    plsc.subcore_barrier()
    pltpu.sync_copy(agg_sh.at[pl.ds(sid * RPS, RPS)],
                    out.at[cid].at[pl.ds(sid * RPS, RPS)])


def _mm_body(x_ref, w_ref, o_ref):
    o_ref[...] = jnp.dot(x_ref[...], w_ref[...],
                         preferred_element_type=jnp.float32)


def _prep_body(h_ref, degp_ref, dis_ref, hs_ref):
    deg = degp_ref[0, :N, 0:1] + degp_ref[1, :N, 0:1] + 1.0
    dis = lax.rsqrt(deg)
    dis_ref[...] = dis
    hs_ref[...] = h_ref[...] * dis


def _norm_relu(conv):
    nrm = jnp.sqrt(jnp.sum(conv * conv, axis=1, keepdims=True))
    return jnp.maximum(conv / jnp.maximum(nrm, 1e-12), 0.0)


def _layer_body(aggp_ref, hs_ref, dis_ref, b_ref, w_ref, hsn_ref):
    dis = dis_ref[...]
    s = aggp_ref[0, :N, :] + aggp_ref[1, :N, :] + hs_ref[...]
    o = _norm_relu(dis * s + b_ref[...])
    hsn_ref[...] = jnp.dot(o, w_ref[...],
                           preferred_element_type=jnp.float32) * dis


def _final_body(aggp_ref, hs_ref, dis_ref, b_ref, batch_ref,
                wf1_ref, bf1_ref, wf2_ref, bf2_ref, out_ref):
    dis = dis_ref[...]
    s = aggp_ref[0, :N, :] + aggp_ref[1, :N, :] + hs_ref[...]
    o = _norm_relu(dis * s + b_ref[...])
    seg = lax.broadcasted_iota(jnp.int32, (G, 1), 0)
    mask = jnp.where(batch_ref[...] == seg, 1.0, 0.0)
    pooled = jnp.dot(mask, o, preferred_element_type=jnp.float32)
    hmid = jnp.maximum(
        jnp.dot(pooled, wf1_ref[...], preferred_element_type=jnp.float32)
        + bf1_ref[...], 0.0)
    logits = (jnp.dot(hmid, wf2_ref[...], preferred_element_type=jnp.float32)
              + bf2_ref[...])
    m = jnp.max(logits, axis=1, keepdims=True)
    lse = m + jnp.log(jnp.sum(jnp.exp(logits - m), axis=1, keepdims=True))
    out_ref[...] = logits - lse


def _f32(shape):
    return jax.ShapeDtypeStruct(shape, jnp.float32)


def kernel(x, edge_index, batch, edge_weights,
           W1, b1, W2, b2, W3, b3, Wf1, bf1, Wf2, bf2):
    src2d = edge_index[0].reshape(NW * NCHUNK, B)
    dst2d = edge_index[1].reshape(NW * NCHUNK, B)
    ew2d = edge_weights.reshape(NW * NCHUNK, B)
    batch2d = batch.reshape(1, N)

    degp = _deg_call(dst2d, ew2d)
    h1 = pl.pallas_call(_mm_body, out_shape=_f32((N, H)))(x, W1)
    dis, hs1 = pl.pallas_call(
        _prep_body, out_shape=[_f32((N, 1)), _f32((N, H))])(h1, degp)

    layer = pl.pallas_call(_layer_body, out_shape=_f32((N, H)))
    aggp1 = _agg_call(hs1, src2d, dst2d, ew2d)
    hs2 = layer(aggp1, hs1, dis, b1.reshape(1, H), W2)
    aggp2 = _agg_call(hs2, src2d, dst2d, ew2d)
    hs3 = layer(aggp2, hs2, dis, b2.reshape(1, H), W3)
    aggp3 = _agg_call(hs3, src2d, dst2d, ew2d)

    return pl.pallas_call(_final_body, out_shape=_f32((G, C)))(
        aggp3, hs3, dis, b3.reshape(1, H), batch2d,
        Wf1, bf1.reshape(1, H), Wf2, bf2.reshape(1, C))


# speedup vs baseline: 20.8389x; 20.8389x over previous
import functools

import jax
import jax.numpy as jnp
from jax import lax
from jax.experimental import pallas as pl
from jax.experimental.pallas import tpu as pltpu
from jax.experimental.pallas import tpu_sc as plsc

N = 10000
E = 320000
F_IN = 128
H = 32
G = 64
C = 10

NP = 10240
NCORE = 2
NSUB = 16
NW = NCORE * NSUB
EPW = E // NW
B = 80
NCHUNK = EPW // B
RPS = NP // NSUB

_mesh = plsc.VectorSubcoreMesh(core_axis_name="c", subcore_axis_name="s")


def _zero_shared(buf, sh, sid, width16):
    zv = jnp.zeros((16,), jnp.float32)
    for r in range(B):
        for k in range(width16):
            buf[r, pl.ds(16 * k, 16)] = zv
    @pl.loop(0, RPS // B)
    def _(k):
        pltpu.sync_copy(buf, sh.at[pl.ds(sid * RPS + k * B, B)])


@functools.partial(
    pl.kernel, mesh=_mesh,
    out_type=jax.ShapeDtypeStruct((NCORE, NP, 16), jnp.float32),
    compiler_params=pltpu.CompilerParams(use_tc_tiling_on_sc=False),
    scratch_types=[
        pltpu.VMEM((EPW,), jnp.int32),
        pltpu.VMEM((EPW,), jnp.float32),
        pltpu.VMEM((B,), jnp.int32),
        pltpu.VMEM((B, 16), jnp.float32),
        pltpu.VMEM_SHARED((NP, 16), jnp.float32),
    ],
)
def _deg_call(dst1d, ew1d, out, dst_v, ew_v, dstc, rows, deg_sh):
    cid = lax.axis_index("c")
    sid = lax.axis_index("s")
    wid = cid * NSUB + sid
    _zero_shared(rows, deg_sh, sid, 1)
    plsc.subcore_barrier()
    pltpu.sync_copy(dst1d.at[pl.ds(wid * EPW, EPW)], dst_v)
    pltpu.sync_copy(ew1d.at[pl.ds(wid * EPW, EPW)], ew_v)

    @pl.loop(0, NCHUNK)
    def _(ci):
        off = ci * B
        for j in range(B // 16):
            dstc[pl.ds(16 * j, 16)] = dst_v[pl.ds(off + 16 * j, 16)]
            wv = ew_v[pl.ds(off + 16 * j, 16)]
            for e in range(16):
                rows[16 * j + e, pl.ds(0, 16)] = jnp.full((16,), wv[e],
                                                          jnp.float32)
        pltpu.sync_copy(rows, deg_sh.at[dstc], add=True)

    plsc.subcore_barrier()
    pltpu.sync_copy(deg_sh.at[pl.ds(sid * RPS, RPS)],
                    out.at[cid].at[pl.ds(sid * RPS, RPS)])


@functools.partial(
    pl.kernel, mesh=_mesh,
    out_type=jax.ShapeDtypeStruct((NCORE, NP, H), jnp.float32),
    compiler_params=pltpu.CompilerParams(use_tc_tiling_on_sc=False),
    scratch_types=[
        pltpu.VMEM((EPW,), jnp.int32),
        pltpu.VMEM((EPW,), jnp.int32),
        pltpu.VMEM((EPW,), jnp.float32),
        pltpu.VMEM((B,), jnp.int32),
        pltpu.VMEM((B,), jnp.int32),
        pltpu.VMEM((B, H), jnp.float32),
        pltpu.VMEM_SHARED((NP, H), jnp.float32),
    ],
)
def _agg_call(hs, src1d, dst1d, ew1d, out,
              src_v, dst_v, ew_v, srcc, dstc, rows, agg_sh):
    cid = lax.axis_index("c")
    sid = lax.axis_index("s")
    wid = cid * NSUB + sid
    _zero_shared(rows, agg_sh, sid, H // 16)
    plsc.subcore_barrier()
    pltpu.sync_copy(src1d.at[pl.ds(wid * EPW, EPW)], src_v)
    pltpu.sync_copy(dst1d.at[pl.ds(wid * EPW, EPW)], dst_v)
    pltpu.sync_copy(ew1d.at[pl.ds(wid * EPW, EPW)], ew_v)

    @pl.loop(0, NCHUNK)
    def _(ci):
        off = ci * B
        for j in range(B // 16):
            srcc[pl.ds(16 * j, 16)] = src_v[pl.ds(off + 16 * j, 16)]
        pltpu.sync_copy(hs.at[srcc], rows)
        for j in range(B // 16):
            dstc[pl.ds(16 * j, 16)] = dst_v[pl.ds(off + 16 * j, 16)]
            wv = ew_v[pl.ds(off + 16 * j, 16)]
            for e in range(16):
                r = 16 * j + e
                wb = jnp.full((16,), wv[e], jnp.float32)
                rows[r, pl.ds(0, 16)] = rows[r, pl.ds(0, 16)] * wb
                rows[r, pl.ds(16, 16)] = rows[r, pl.ds(16, 16)] * wb
        pltpu.sync_copy(rows, agg_sh.at[dstc], add=True)

    plsc.subcore_barrier()
    pltpu.sync_copy(agg_sh.at[pl.ds(sid * RPS, RPS)],
                    out.at[cid].at[pl.ds(sid * RPS, RPS)])


def _mm_body(x_ref, w_ref, o_ref):
    o_ref[...] = jnp.dot(x_ref[...], w_ref[...],
                         preferred_element_type=jnp.float32)


def _prep_body(h_ref, degp_ref, dis_ref, hs_ref):
    deg = degp_ref[0, :N, 0:1] + degp_ref[1, :N, 0:1] + 1.0
    dis = lax.rsqrt(deg)
    dis_ref[...] = dis
    hs_ref[:N, :] = h_ref[...] * dis


def _norm_relu(conv):
    nrm = jnp.sqrt(jnp.sum(conv * conv, axis=1, keepdims=True))
    return jnp.maximum(conv / jnp.maximum(nrm, 1e-12), 0.0)


def _layer_body(aggp_ref, hs_ref, dis_ref, b_ref, w_ref, hsn_ref):
    dis = dis_ref[...]
    s = aggp_ref[0, :N, :] + aggp_ref[1, :N, :] + hs_ref[:N, :]
    o = _norm_relu(dis * s + b_ref[...])
    hsn_ref[:N, :] = jnp.dot(o, w_ref[...],
                             preferred_element_type=jnp.float32) * dis


def _final_body(aggp_ref, hs_ref, dis_ref, b_ref, batch_ref,
                wf1_ref, bf1_ref, wf2_ref, bf2_ref, out_ref):
    dis = dis_ref[...]
    s = aggp_ref[0, :N, :] + aggp_ref[1, :N, :] + hs_ref[:N, :]
    o = _norm_relu(dis * s + b_ref[...])
    seg = lax.broadcasted_iota(jnp.int32, (G, 1), 0)
    mask = jnp.where(batch_ref[...] == seg, 1.0, 0.0)
    pooled = jnp.dot(mask, o, preferred_element_type=jnp.float32)
    hmid = jnp.maximum(
        jnp.dot(pooled, wf1_ref[...], preferred_element_type=jnp.float32)
        + bf1_ref[...], 0.0)
    logits = (jnp.dot(hmid, wf2_ref[...], preferred_element_type=jnp.float32)
              + bf2_ref[...])
    m = jnp.max(logits, axis=1, keepdims=True)
    lse = m + jnp.log(jnp.sum(jnp.exp(logits - m), axis=1, keepdims=True))
    out_ref[...] = logits - lse


def _f32(shape):
    return jax.ShapeDtypeStruct(shape, jnp.float32)


def kernel(x, edge_index, batch, edge_weights,
           W1, b1, W2, b2, W3, b3, Wf1, bf1, Wf2, bf2):
    src1d = edge_index[0]
    dst1d = edge_index[1]
    batch2d = batch.reshape(1, N)

    degp = _deg_call(dst1d, edge_weights)
    h1 = pl.pallas_call(_mm_body, out_shape=_f32((N, H)))(x, W1)
    dis, hs1 = pl.pallas_call(
        _prep_body, out_shape=[_f32((N, 1)), _f32((NP, H))])(h1, degp)

    layer = pl.pallas_call(_layer_body, out_shape=_f32((NP, H)))
    aggp1 = _agg_call(hs1, src1d, dst1d, edge_weights)
    hs2 = layer(aggp1, hs1, dis, b1.reshape(1, H), W2)
    aggp2 = _agg_call(hs2, src1d, dst1d, edge_weights)
    hs3 = layer(aggp2, hs2, dis, b2.reshape(1, H), W3)
    aggp3 = _agg_call(hs3, src1d, dst1d, edge_weights)

    return pl.pallas_call(_final_body, out_shape=_f32((G, C)))(
        aggp3, hs3, dis, b3.reshape(1, H), batch2d,
        Wf1, bf1.reshape(1, H), Wf2, bf2.reshape(1, C))


# speedup vs baseline: 30.8336x; 1.4796x over previous
import functools

import jax
import jax.numpy as jnp
from jax import lax
from jax.experimental import pallas as pl
from jax.experimental.pallas import tpu as pltpu
from jax.experimental.pallas import tpu_sc as plsc

N = 10000
E = 320000
F_IN = 128
H = 32
G = 64
C = 10

NP = 10240
NCORE = 2
NSUB = 16
NW = NCORE * NSUB
EPW = E // NW
B = 80
NCHUNK = EPW // B
RPS = NP // NSUB

_mesh = plsc.VectorSubcoreMesh(core_axis_name="c", subcore_axis_name="s")


def _zero_shared(buf, sh, sid, width16):
    zv = jnp.zeros((16,), jnp.float32)
    for r in range(B):
        for k in range(width16):
            buf[r, pl.ds(16 * k, 16)] = zv
    @pl.loop(0, RPS // B)
    def _(k):
        pltpu.sync_copy(buf, sh.at[pl.ds(sid * RPS + k * B, B)])


@functools.partial(
    pl.kernel, mesh=_mesh,
    out_type=jax.ShapeDtypeStruct((NCORE, NP, 16), jnp.float32),
    compiler_params=pltpu.CompilerParams(use_tc_tiling_on_sc=False),
    scratch_types=[
        pltpu.VMEM((EPW,), jnp.int32),
        pltpu.VMEM((EPW,), jnp.float32),
        pltpu.VMEM((B,), jnp.int32),
        pltpu.VMEM((B, 16), jnp.float32),
        pltpu.VMEM_SHARED((NP, 16), jnp.float32),
    ],
)
def _deg_call(dst1d, ew1d, out, dst_v, ew_v, dstc, rows, deg_sh):
    cid = lax.axis_index("c")
    sid = lax.axis_index("s")
    wid = cid * NSUB + sid
    _zero_shared(rows, deg_sh, sid, 1)
    plsc.subcore_barrier()
    pltpu.sync_copy(dst1d.at[pl.ds(wid * EPW, EPW)], dst_v)
    pltpu.sync_copy(ew1d.at[pl.ds(wid * EPW, EPW)], ew_v)

    @pl.loop(0, NCHUNK)
    def _(ci):
        off = ci * B
        for j in range(B // 16):
            dstc[pl.ds(16 * j, 16)] = dst_v[pl.ds(off + 16 * j, 16)]
            wv = ew_v[pl.ds(off + 16 * j, 16)]
            for e in range(16):
                rows[16 * j + e, pl.ds(0, 16)] = jnp.full((16,), wv[e],
                                                          jnp.float32)
        pltpu.sync_copy(rows, deg_sh.at[dstc], add=True)

    plsc.subcore_barrier()
    pltpu.sync_copy(deg_sh.at[pl.ds(sid * RPS, RPS)],
                    out.at[cid].at[pl.ds(sid * RPS, RPS)])


def _build_idx(buf, vec, off):
    for j in range(B // 16):
        buf[pl.ds(16 * j, 16)] = vec[pl.ds(off + 16 * j, 16)]


def _scale_scatter(rows, dstc, dst_v, ew_v, off, sh):
    for j in range(B // 16):
        dstc[pl.ds(16 * j, 16)] = dst_v[pl.ds(off + 16 * j, 16)]
        wv = ew_v[pl.ds(off + 16 * j, 16)]
        for e in range(16):
            r = 16 * j + e
            wb = jnp.full((16,), wv[e], jnp.float32)
            rows[r, pl.ds(0, 16)] = rows[r, pl.ds(0, 16)] * wb
            rows[r, pl.ds(16, 16)] = rows[r, pl.ds(16, 16)] * wb
    pltpu.sync_copy(rows, sh.at[dstc], add=True)


@functools.partial(
    pl.kernel, mesh=_mesh,
    out_type=jax.ShapeDtypeStruct((NCORE, NP, H), jnp.float32),
    compiler_params=pltpu.CompilerParams(use_tc_tiling_on_sc=False),
    scratch_types=[
        pltpu.VMEM((EPW,), jnp.int32),
        pltpu.VMEM((EPW,), jnp.int32),
        pltpu.VMEM((EPW,), jnp.float32),
        pltpu.VMEM((B,), jnp.int32),
        pltpu.VMEM((B,), jnp.int32),
        pltpu.VMEM((B,), jnp.int32),
        pltpu.VMEM((B, H), jnp.float32),
        pltpu.VMEM((B, H), jnp.float32),
        pltpu.VMEM_SHARED((NP, H), jnp.float32),
        pltpu.SemaphoreType.DMA,
        pltpu.SemaphoreType.DMA,
    ],
)
def _agg_call(hs, src1d, dst1d, ew1d, out,
              src_v, dst_v, ew_v, srcc0, srcc1, dstc,
              rows0, rows1, agg_sh, gsem0, gsem1):
    cid = lax.axis_index("c")
    sid = lax.axis_index("s")
    wid = cid * NSUB + sid
    _zero_shared(rows0, agg_sh, sid, H // 16)
    plsc.subcore_barrier()
    pltpu.sync_copy(src1d.at[pl.ds(wid * EPW, EPW)], src_v)
    pltpu.sync_copy(dst1d.at[pl.ds(wid * EPW, EPW)], dst_v)
    pltpu.sync_copy(ew1d.at[pl.ds(wid * EPW, EPW)], ew_v)

    _build_idx(srcc0, src_v, 0)
    pltpu.async_copy(hs.at[srcc0], rows0, gsem0)

    @pl.loop(0, (NCHUNK - 1) // 2)
    def _(t):
        off = 2 * t * B
        _build_idx(srcc1, src_v, off + B)
        pltpu.async_copy(hs.at[srcc1], rows1, gsem1)
        pltpu.make_async_copy(hs.at[srcc0], rows0, gsem0).wait()
        _scale_scatter(rows0, dstc, dst_v, ew_v, off, agg_sh)
        _build_idx(srcc0, src_v, off + 2 * B)
        pltpu.async_copy(hs.at[srcc0], rows0, gsem0)
        pltpu.make_async_copy(hs.at[srcc1], rows1, gsem1).wait()
        _scale_scatter(rows1, dstc, dst_v, ew_v, off + B, agg_sh)

    pltpu.make_async_copy(hs.at[srcc0], rows0, gsem0).wait()
    _scale_scatter(rows0, dstc, dst_v, ew_v, (NCHUNK - 1) * B, agg_sh)

    plsc.subcore_barrier()
    pltpu.sync_copy(agg_sh.at[pl.ds(sid * RPS, RPS)],
                    out.at[cid].at[pl.ds(sid * RPS, RPS)])


def _mm_body(x_ref, w_ref, o_ref):
    o_ref[...] = jnp.dot(x_ref[...], w_ref[...],
                         preferred_element_type=jnp.float32)


def _prep_body(h_ref, degp_ref, dis_ref, hs_ref):
    deg = degp_ref[0, :N, 0:1] + degp_ref[1, :N, 0:1] + 1.0
    dis = lax.rsqrt(deg)
    dis_ref[...] = dis
    hs_ref[:N, :] = h_ref[...] * dis


def _norm_relu(conv):
    nrm = jnp.sqrt(jnp.sum(conv * conv, axis=1, keepdims=True))
    return jnp.maximum(conv / jnp.maximum(nrm, 1e-12), 0.0)


def _layer_body(aggp_ref, hs_ref, dis_ref, b_ref, w_ref, hsn_ref):
    dis = dis_ref[...]
    s = aggp_ref[0, :N, :] + aggp_ref[1, :N, :] + hs_ref[:N, :]
    o = _norm_relu(dis * s + b_ref[...])
    hsn_ref[:N, :] = jnp.dot(o, w_ref[...],
                             preferred_element_type=jnp.float32) * dis


def _final_body(aggp_ref, hs_ref, dis_ref, b_ref, batch_ref,
                wf1_ref, bf1_ref, wf2_ref, bf2_ref, out_ref):
    dis = dis_ref[...]
    s = aggp_ref[0, :N, :] + aggp_ref[1, :N, :] + hs_ref[:N, :]
    o = _norm_relu(dis * s + b_ref[...])
    seg = lax.broadcasted_iota(jnp.int32, (G, 1), 0)
    mask = jnp.where(batch_ref[...] == seg, 1.0, 0.0)
    pooled = jnp.dot(mask, o, preferred_element_type=jnp.float32)
    hmid = jnp.maximum(
        jnp.dot(pooled, wf1_ref[...], preferred_element_type=jnp.float32)
        + bf1_ref[...], 0.0)
    logits = (jnp.dot(hmid, wf2_ref[...], preferred_element_type=jnp.float32)
              + bf2_ref[...])
    m = jnp.max(logits, axis=1, keepdims=True)
    lse = m + jnp.log(jnp.sum(jnp.exp(logits - m), axis=1, keepdims=True))
    out_ref[...] = logits - lse


def _f32(shape):
    return jax.ShapeDtypeStruct(shape, jnp.float32)


def kernel(x, edge_index, batch, edge_weights,
           W1, b1, W2, b2, W3, b3, Wf1, bf1, Wf2, bf2):
    src1d = edge_index[0]
    dst1d = edge_index[1]
    batch2d = batch.reshape(1, N)

    degp = _deg_call(dst1d, edge_weights)
    h1 = pl.pallas_call(_mm_body, out_shape=_f32((N, H)))(x, W1)
    dis, hs1 = pl.pallas_call(
        _prep_body, out_shape=[_f32((N, 1)), _f32((NP, H))])(h1, degp)

    layer = pl.pallas_call(_layer_body, out_shape=_f32((NP, H)))
    aggp1 = _agg_call(hs1, src1d, dst1d, edge_weights)
    hs2 = layer(aggp1, hs1, dis, b1.reshape(1, H), W2)
    aggp2 = _agg_call(hs2, src1d, dst1d, edge_weights)
    hs3 = layer(aggp2, hs2, dis, b2.reshape(1, H), W3)
    aggp3 = _agg_call(hs3, src1d, dst1d, edge_weights)

    return pl.pallas_call(_final_body, out_shape=_f32((G, C)))(
        aggp3, hs3, dis, b3.reshape(1, H), batch2d,
        Wf1, bf1.reshape(1, H), Wf2, bf2.reshape(1, C))


# speedup vs baseline: 32.0585x; 1.0397x over previous
import functools

import jax
import jax.numpy as jnp
from jax import lax
from jax.experimental import pallas as pl
from jax.experimental.pallas import tpu as pltpu
from jax.experimental.pallas import tpu_sc as plsc

N = 10000
E = 320000
F_IN = 128
H = 32
G = 64
C = 10

NP = 10240
NCORE = 2
NSUB = 16
NW = NCORE * NSUB
EPW = E // NW
B = 80
NCHUNK = EPW // B
RPS = NP // NSUB

_mesh = plsc.VectorSubcoreMesh(core_axis_name="c", subcore_axis_name="s")


def _zero_shared(buf, sh, sid, width16):
    zv = jnp.zeros((16,), jnp.float32)
    for r in range(B):
        for k in range(width16):
            buf[r, pl.ds(16 * k, 16)] = zv
    @pl.loop(0, RPS // B)
    def _(k):
        pltpu.sync_copy(buf, sh.at[pl.ds(sid * RPS + k * B, B)])


@functools.partial(
    pl.kernel, mesh=_mesh,
    out_type=jax.ShapeDtypeStruct((NCORE, NP, 16), jnp.float32),
    compiler_params=pltpu.CompilerParams(use_tc_tiling_on_sc=False),
    scratch_types=[
        pltpu.VMEM((EPW,), jnp.int32),
        pltpu.VMEM((EPW,), jnp.float32),
        pltpu.VMEM((B,), jnp.int32),
        pltpu.VMEM((B,), jnp.int32),
        pltpu.VMEM((B, 16), jnp.float32),
        pltpu.VMEM((B, 16), jnp.float32),
        pltpu.VMEM_SHARED((NP, 16), jnp.float32),
        pltpu.SemaphoreType.DMA,
        pltpu.SemaphoreType.DMA,
    ],
)
def _deg_call(dst1d, ew1d, out, dst_v, ew_v, dstc0, dstc1,
              rows0, rows1, deg_sh, ssem0, ssem1):
    cid = lax.axis_index("c")
    sid = lax.axis_index("s")
    wid = cid * NSUB + sid
    _zero_shared(rows0, deg_sh, sid, 1)
    plsc.subcore_barrier()
    pltpu.sync_copy(dst1d.at[pl.ds(wid * EPW, EPW)], dst_v)
    pltpu.sync_copy(ew1d.at[pl.ds(wid * EPW, EPW)], ew_v)

    def build(rows, dstc, off):
        for j in range(B // 16):
            dstc[pl.ds(16 * j, 16)] = dst_v[pl.ds(off + 16 * j, 16)]
            wv = ew_v[pl.ds(off + 16 * j, 16)]
            for e in range(16):
                rows[16 * j + e, pl.ds(0, 16)] = jnp.full((16,), wv[e],
                                                          jnp.float32)

    build(rows0, dstc0, 0)
    pltpu.async_copy(rows0, deg_sh.at[dstc0], ssem0, add=True)
    build(rows1, dstc1, B)
    pltpu.async_copy(rows1, deg_sh.at[dstc1], ssem1, add=True)

    @pl.loop(0, (NCHUNK - 3) // 2)
    def _(t):
        off = (2 * t + 2) * B
        pltpu.make_async_copy(rows0, deg_sh.at[dstc0], ssem0).wait()
        build(rows0, dstc0, off)
        pltpu.async_copy(rows0, deg_sh.at[dstc0], ssem0, add=True)
        pltpu.make_async_copy(rows1, deg_sh.at[dstc1], ssem1).wait()
        build(rows1, dstc1, off + B)
        pltpu.async_copy(rows1, deg_sh.at[dstc1], ssem1, add=True)

    pltpu.make_async_copy(rows0, deg_sh.at[dstc0], ssem0).wait()
    build(rows0, dstc0, (NCHUNK - 1) * B)
    pltpu.async_copy(rows0, deg_sh.at[dstc0], ssem0, add=True)
    pltpu.make_async_copy(rows0, deg_sh.at[dstc0], ssem0).wait()
    pltpu.make_async_copy(rows1, deg_sh.at[dstc1], ssem1).wait()

    plsc.subcore_barrier()
    pltpu.sync_copy(deg_sh.at[pl.ds(sid * RPS, RPS)],
                    out.at[cid].at[pl.ds(sid * RPS, RPS)])


def _scale_scatter(rows, dstc, dst_v, ew_v, off, sh):
    for j in range(B // 16):
        dstc[pl.ds(16 * j, 16)] = dst_v[pl.ds(off + 16 * j, 16)]
        wv = ew_v[pl.ds(off + 16 * j, 16)]
        for e in range(16):
            r = 16 * j + e
            wb = jnp.full((16,), wv[e], jnp.float32)
            rows[r, pl.ds(0, 16)] = rows[r, pl.ds(0, 16)] * wb
            rows[r, pl.ds(16, 16)] = rows[r, pl.ds(16, 16)] * wb
    pltpu.sync_copy(rows, sh.at[dstc], add=True)


@functools.partial(
    pl.kernel, mesh=_mesh,
    out_type=jax.ShapeDtypeStruct((NCORE, NP, H), jnp.float32),
    compiler_params=pltpu.CompilerParams(use_tc_tiling_on_sc=False),
    scratch_types=[
        pltpu.VMEM((EPW,), jnp.int32),
        pltpu.VMEM((EPW,), jnp.int32),
        pltpu.VMEM((EPW,), jnp.float32),
        pltpu.VMEM((B,), jnp.int32),
        pltpu.VMEM((B, H), jnp.float32),
        pltpu.VMEM((B, H), jnp.float32),
        pltpu.VMEM_SHARED((NP, H), jnp.float32),
        pltpu.SemaphoreType.DMA,
        pltpu.SemaphoreType.DMA,
    ],
)
def _agg_call(hs, src1d, dst1d, ew1d, out,
              src_v, dst_v, ew_v, dstc,
              rows0, rows1, agg_sh, gsem0, gsem1):
    cid = lax.axis_index("c")
    sid = lax.axis_index("s")
    wid = cid * NSUB + sid
    _zero_shared(rows0, agg_sh, sid, H // 16)
    plsc.subcore_barrier()
    pltpu.sync_copy(src1d.at[pl.ds(wid * EPW, EPW)], src_v)
    pltpu.sync_copy(dst1d.at[pl.ds(wid * EPW, EPW)], dst_v)
    pltpu.sync_copy(ew1d.at[pl.ds(wid * EPW, EPW)], ew_v)

    pltpu.async_copy(hs.at[src_v.at[pl.ds(0, B)]], rows0, gsem0)

    @pl.loop(0, (NCHUNK - 1) // 2)
    def _(t):
        off = 2 * t * B
        pltpu.async_copy(hs.at[src_v.at[pl.ds(off + B, B)]], rows1, gsem1)
        pltpu.make_async_copy(hs.at[src_v.at[pl.ds(off, B)]],
                              rows0, gsem0).wait()
        _scale_scatter(rows0, dstc, dst_v, ew_v, off, agg_sh)
        pltpu.async_copy(hs.at[src_v.at[pl.ds(off + 2 * B, B)]], rows0, gsem0)
        pltpu.make_async_copy(hs.at[src_v.at[pl.ds(off + B, B)]],
                              rows1, gsem1).wait()
        _scale_scatter(rows1, dstc, dst_v, ew_v, off + B, agg_sh)

    pltpu.make_async_copy(hs.at[src_v.at[pl.ds(0, B)]], rows0, gsem0).wait()
    _scale_scatter(rows0, dstc, dst_v, ew_v, (NCHUNK - 1) * B, agg_sh)

    plsc.subcore_barrier()
    pltpu.sync_copy(agg_sh.at[pl.ds(sid * RPS, RPS)],
                    out.at[cid].at[pl.ds(sid * RPS, RPS)])


def _mm_body(x_ref, w_ref, o_ref):
    o_ref[...] = jnp.dot(x_ref[...], w_ref[...],
                         preferred_element_type=jnp.float32)


def _prep_body(h_ref, degp_ref, dis_ref, hs_ref):
    deg = degp_ref[0, :N, 0:1] + degp_ref[1, :N, 0:1] + 1.0
    dis = lax.rsqrt(deg)
    dis_ref[...] = dis
    hs_ref[:N, :] = h_ref[...] * dis


def _norm_relu(conv):
    nrm = jnp.sqrt(jnp.sum(conv * conv, axis=1, keepdims=True))
    return jnp.maximum(conv / jnp.maximum(nrm, 1e-12), 0.0)


def _layer_body(aggp_ref, hs_ref, dis_ref, b_ref, w_ref, hsn_ref):
    dis = dis_ref[...]
    s = aggp_ref[0, :N, :] + aggp_ref[1, :N, :] + hs_ref[:N, :]
    o = _norm_relu(dis * s + b_ref[...])
    hsn_ref[:N, :] = jnp.dot(o, w_ref[...],
                             preferred_element_type=jnp.float32) * dis


def _final_body(aggp_ref, hs_ref, dis_ref, b_ref, batch_ref,
                wf1_ref, bf1_ref, wf2_ref, bf2_ref, out_ref):
    dis = dis_ref[...]
    s = aggp_ref[0, :N, :] + aggp_ref[1, :N, :] + hs_ref[:N, :]
    o = _norm_relu(dis * s + b_ref[...])
    seg = lax.broadcasted_iota(jnp.int32, (G, 1), 0)
    mask = jnp.where(batch_ref[...] == seg, 1.0, 0.0)
    pooled = jnp.dot(mask, o, preferred_element_type=jnp.float32)
    hmid = jnp.maximum(
        jnp.dot(pooled, wf1_ref[...], preferred_element_type=jnp.float32)
        + bf1_ref[...], 0.0)
    logits = (jnp.dot(hmid, wf2_ref[...], preferred_element_type=jnp.float32)
              + bf2_ref[...])
    m = jnp.max(logits, axis=1, keepdims=True)
    lse = m + jnp.log(jnp.sum(jnp.exp(logits - m), axis=1, keepdims=True))
    out_ref[...] = logits - lse


def _f32(shape):
    return jax.ShapeDtypeStruct(shape, jnp.float32)


def kernel(x, edge_index, batch, edge_weights,
           W1, b1, W2, b2, W3, b3, Wf1, bf1, Wf2, bf2):
    src1d = edge_index[0]
    dst1d = edge_index[1]
    batch2d = batch.reshape(1, N)

    degp = _deg_call(dst1d, edge_weights)
    h1 = pl.pallas_call(_mm_body, out_shape=_f32((N, H)))(x, W1)
    dis, hs1 = pl.pallas_call(
        _prep_body, out_shape=[_f32((N, 1)), _f32((NP, H))])(h1, degp)

    layer = pl.pallas_call(_layer_body, out_shape=_f32((NP, H)))
    aggp1 = _agg_call(hs1, src1d, dst1d, edge_weights)
    hs2 = layer(aggp1, hs1, dis, b1.reshape(1, H), W2)
    aggp2 = _agg_call(hs2, src1d, dst1d, edge_weights)
    hs3 = layer(aggp2, hs2, dis, b2.reshape(1, H), W3)
    aggp3 = _agg_call(hs3, src1d, dst1d, edge_weights)

    return pl.pallas_call(_final_body, out_shape=_f32((G, C)))(
        aggp3, hs3, dis, b3.reshape(1, H), batch2d,
        Wf1, bf1.reshape(1, H), Wf2, bf2.reshape(1, C))
